# Initial kernel scaffold; baseline (speedup 1.0000x reference)
import jax
import jax.numpy as jnp
from jax.experimental import pallas as pl


def kernel(input, table):
    raise NotImplementedError("write your pallas kernel here")


# speedup vs baseline: 1.4259x; 1.4259x over previous
import functools
import math

import jax
import jax.numpy as jnp
from jax import lax
from jax.experimental import pallas as pl
from jax.experimental.pallas import tpu as pltpu
from jax.experimental.pallas import tpu_sc as plsc

D = 32
SCALE = math.sqrt(D)
NC, NS = 2, 16
NW = NC * NS
W = 50
S = 16384
SH = S // 128
SPW = S // NW
JJ = SPW // 128


@functools.cache
def _emb():
    mesh = plsc.VectorSubcoreMesh(core_axis_name="c", subcore_axis_name="s")

    @functools.partial(
        pl.kernel, mesh=mesh,
        out_type=jax.ShapeDtypeStruct((W, D // 8, SH, 8, 128), jnp.float32),
        scratch_types=[
            pltpu.VMEM((2, JJ, 128), jnp.int32),
            pltpu.VMEM((2, SPW, D), jnp.float32),
            pltpu.VMEM((2, D // 8, JJ, 8, 128), jnp.float32),
            pltpu.SemaphoreType.DMA,
            pltpu.SemaphoreType.DMA,
            pltpu.SemaphoreType.DMA,
            pltpu.SemaphoreType.DMA,
        ],
        compiler_params=pltpu.CompilerParams(use_tc_tiling_on_sc=False,
                                             needs_layout_passes=False),
    )
    def emb(table_hbm, idxt_hbm, out_hbm, idx_v, rows_v, stage_v,
            g0, g1, ws0, ws1):
        wid = lax.axis_index("s") * NC + lax.axis_index("c")
        sbase = wid * SPW
        gsem = (g0, g1)
        wsem = (ws0, ws1)
        iota16 = lax.iota(jnp.int32, 16)

        def fire(w, b):
            for j in range(JJ):
                pltpu.sync_copy(idxt_hbm.at[w, pl.ds(sbase + j * 128, 128)],
                                idx_v.at[b, j])
            for j in range(JJ):
                pltpu.async_copy(table_hbm.at[idx_v.at[b, j]],
                                 rows_v.at[b, pl.ds(j * 128, 128)], gsem[b])

        def drain_gather(b):
            for j in range(JJ):
                pltpu.make_async_copy(table_hbm.at[idx_v.at[b, j]],
                                      rows_v.at[b, pl.ds(j * 128, 128)],
                                      gsem[b]).wait()

        def transpose_scale(b):
            rows2d = rows_v.at[b]

            def body(g, carry):
                shl = lax.shift_right_logical(g, 3)
                soff = lax.bitwise_and(g, 7) * 16
                svec = iota16 + jnp.full((16,), g * 16, jnp.int32)
                for e in range(D):
                    evec = jnp.full((16,), e, jnp.int32)
                    vals = plsc.load_gather(rows2d, [svec, evec])
                    stage_v[b, e // 8, shl, e % 8, pl.ds(soff, 16)] = (
                        vals * SCALE)
                return carry

            lax.fori_loop(0, SPW // 16, body, 0)

        def fire_write(w, b):
            for e_hi in range(D // 8):
                pltpu.async_copy(stage_v.at[b, e_hi],
                                 out_hbm.at[w, e_hi, pl.ds(wid * JJ, JJ)],
                                 wsem[b])

        def drain_write(w, b):
            for e_hi in range(D // 8):
                pltpu.make_async_copy(stage_v.at[b, e_hi],
                                      out_hbm.at[w, e_hi, pl.ds(wid * JJ, JJ)],
                                      wsem[b]).wait()

        fire(0, 0)

        def outer(k, carry):
            for b in (0, 1):
                w = 2 * k + b
                if b == 0:
                    fire(w + 1, 1)
                else:
                    @pl.when(k < W // 2 - 1)
                    def _():
                        fire(w + 1, 0)
                drain_gather(b)

                @pl.when(k > 0)
                def _():
                    drain_write(w, b)

                transpose_scale(b)
                fire_write(w, b)
            return carry

        lax.fori_loop(0, W // 2, outer, 0)
        drain_write(W - 2, 0)
        drain_write(W - 1, 1)

    return emb


def kernel(input, table):
    idx_t = input.T
    out5 = _emb()(table, idx_t)
    return out5.transpose(2, 4, 0, 1, 3).reshape(S, W, D)


# speedup vs baseline: 1.5449x; 1.0835x over previous
import functools
import math

import jax
import jax.numpy as jnp
from jax import lax
from jax.experimental import pallas as pl
from jax.experimental.pallas import tpu as pltpu
from jax.experimental.pallas import tpu_sc as plsc

D = 32
SCALE = math.sqrt(D)
NC, NS = 2, 16
NW = NC * NS
W = 50
S = 16384
SH = S // 128
SPW = S // NW
JJ = SPW // 128
RSTR = D


@functools.cache
def _emb():
    mesh = plsc.VectorSubcoreMesh(core_axis_name="c", subcore_axis_name="s")

    @functools.partial(
        pl.kernel, mesh=mesh,
        out_type=jax.ShapeDtypeStruct((W, D // 8, SH, 8, 128), jnp.float32),
        scratch_types=[
            pltpu.VMEM((2, JJ, 128), jnp.int32),
            pltpu.VMEM((2, SPW, RSTR), jnp.float32),
            pltpu.VMEM((2, D // 8, JJ, 8, 128), jnp.float32),
            pltpu.SemaphoreType.DMA,
            pltpu.SemaphoreType.DMA,
            pltpu.SemaphoreType.DMA,
            pltpu.SemaphoreType.DMA,
            pltpu.SemaphoreType.DMA,
            pltpu.SemaphoreType.DMA,
        ],
        compiler_params=pltpu.CompilerParams(use_tc_tiling_on_sc=False,
                                             needs_layout_passes=False),
    )
    def emb(table_hbm, idx3_hbm, out_hbm, idx_v, rows_v, stage_v,
            g0, g1, ws0, ws1, is0, is1):
        wid = lax.axis_index("s") * NC + lax.axis_index("c")
        gsem = (g0, g1)
        wsem = (ws0, ws1)
        isem = (is0, is1)
        iota16 = lax.iota(jnp.int32, 16)

        def fire_idx(w, b):
            pltpu.async_copy(idx3_hbm.at[w, pl.ds(wid * JJ, JJ)],
                             idx_v.at[b], isem[b])

        def wait_idx(w, b):
            pltpu.make_async_copy(idx3_hbm.at[w, pl.ds(wid * JJ, JJ)],
                                  idx_v.at[b], isem[b]).wait()

        def fire_gather(b):
            for j in range(JJ):
                pltpu.async_copy(table_hbm.at[idx_v.at[b, j]],
                                 rows_v.at[b, pl.ds(j * 128, 128)], gsem[b])

        def drain_gather(b):
            for j in range(JJ):
                pltpu.make_async_copy(table_hbm.at[idx_v.at[b, j]],
                                      rows_v.at[b, pl.ds(j * 128, 128)],
                                      gsem[b]).wait()

        def transpose_scale(b):
            rows2d = rows_v.at[b]

            def body(g, carry):
                shl = lax.shift_right_logical(g, 3)
                soff = lax.bitwise_and(g, 7) * 16
                svec = iota16 + jnp.full((16,), g * 16, jnp.int32)
                for e in range(D):
                    evec = jnp.full((16,), e, jnp.int32)
                    vals = plsc.load_gather(rows2d, [svec, evec])
                    stage_v[b, e // 8, shl, e % 8, pl.ds(soff, 16)] = (
                        vals * SCALE)
                return carry

            lax.fori_loop(0, SPW // 16, body, 0)

        def fire_write(w, b):
            for e_hi in range(D // 8):
                pltpu.async_copy(stage_v.at[b, e_hi],
                                 out_hbm.at[w, e_hi, pl.ds(wid * JJ, JJ)],
                                 wsem[b])

        def drain_write(w, b):
            for e_hi in range(D // 8):
                pltpu.make_async_copy(stage_v.at[b, e_hi],
                                      out_hbm.at[w, e_hi, pl.ds(wid * JJ, JJ)],
                                      wsem[b]).wait()

        fire_idx(0, 0)
        wait_idx(0, 0)
        fire_gather(0)
        fire_idx(1, 1)

        def outer(k, carry):
            for b in (0, 1):
                w = 2 * k + b
                drain_gather(b)
                if b == 0:
                    wait_idx(w + 1, 1)
                    fire_gather(1)
                else:
                    @pl.when(k < W // 2 - 1)
                    def _():
                        wait_idx(w + 1, 0)
                        fire_gather(0)

                @pl.when(k < W // 2 - 1)
                def _():
                    fire_idx(w + 2, b)

                @pl.when(k > 0)
                def _():
                    drain_write(w, b)

                transpose_scale(b)
                fire_write(w, b)
            return carry

        lax.fori_loop(0, W // 2, outer, 0)
        drain_write(W - 2, 0)
        drain_write(W - 1, 1)

    return emb


def kernel(input, table):
    idx3 = input.T.reshape(W, SH, 128)
    out5 = _emb()(table, idx3)
    return out5.transpose(2, 4, 0, 1, 3).reshape(S, W, D)


# speedup vs baseline: 2.7557x; 1.7838x over previous
import functools
import math

import jax
import jax.numpy as jnp
from jax import lax
from jax.experimental import pallas as pl
from jax.experimental.pallas import tpu as pltpu
from jax.experimental.pallas import tpu_sc as plsc

D = 32
SCALE = math.sqrt(D)
NC, NS = 2, 16
NW = NC * NS
W = 50
S = 16384
SH = S // 128
SPW = S // NW
JJ = SPW // 128
RSTR = D


@functools.cache
def _emb():
    mesh = plsc.VectorSubcoreMesh(core_axis_name="c", subcore_axis_name="s")

    @functools.partial(
        pl.kernel, mesh=mesh,
        out_type=jax.ShapeDtypeStruct((W, D // 8, SH, 8, 128), jnp.float32),
        scratch_types=[
            pltpu.VMEM((2, JJ, 128), jnp.int32),
            pltpu.VMEM((2, SPW, RSTR), jnp.float32),
            pltpu.VMEM((2, D // 8, JJ + 1, 8, 129), jnp.float32),
            pltpu.SemaphoreType.DMA,
            pltpu.SemaphoreType.DMA,
            pltpu.SemaphoreType.DMA,
            pltpu.SemaphoreType.DMA,
            pltpu.SemaphoreType.DMA,
            pltpu.SemaphoreType.DMA,
        ],
        compiler_params=pltpu.CompilerParams(use_tc_tiling_on_sc=False,
                                             needs_layout_passes=False),
    )
    def emb(table_hbm, idx3_hbm, out_hbm, idx_v, rows_v, stage_v,
            g0, g1, ws0, ws1, is0, is1):
        wid = lax.axis_index("s") * NC + lax.axis_index("c")
        gsem = (g0, g1)
        wsem = (ws0, ws1)
        isem = (is0, is1)
        iota16 = lax.iota(jnp.int32, 16)

        def fire_idx(w, b):
            pltpu.async_copy(idx3_hbm.at[w, pl.ds(wid * JJ, JJ)],
                             idx_v.at[b], isem[b])

        def wait_idx(w, b):
            pltpu.make_async_copy(idx3_hbm.at[w, pl.ds(wid * JJ, JJ)],
                                  idx_v.at[b], isem[b]).wait()

        def fire_gather(b):
            for j in range(JJ):
                pltpu.async_copy(table_hbm.at[idx_v.at[b, j]],
                                 rows_v.at[b, pl.ds(j * 128, 128)], gsem[b])

        def drain_gather(b):
            for j in range(JJ):
                pltpu.make_async_copy(table_hbm.at[idx_v.at[b, j]],
                                      rows_v.at[b, pl.ds(j * 128, 128)],
                                      gsem[b]).wait()

        ehi_lo = lax.shift_right_logical(iota16, 3)
        ehi_hi = ehi_lo + 2
        elo = lax.bitwise_and(iota16, 7)

        def transpose_scale(b):
            stg = stage_v.at[b]

            def body(t, carry):
                shl = jnp.full((16,), lax.shift_right_logical(t, 7),
                               jnp.int32)
                slo = jnp.full((16,), lax.bitwise_and(t, 127), jnp.int32)
                v1 = rows_v[b, t, pl.ds(0, 16)] * SCALE
                v2 = rows_v[b, t, pl.ds(16, 16)] * SCALE
                plsc.store_scatter(stg, [ehi_lo, shl, elo, slo], v1)
                plsc.store_scatter(stg, [ehi_hi, shl, elo, slo], v2)
                return carry

            lax.fori_loop(0, SPW, body, 0)

        def fire_write(w, b):
            for e_hi in range(D // 8):
                for sl in range(JJ):
                    pltpu.async_copy(
                        stage_v.at[b, e_hi, sl, pl.ds(0, 8), pl.ds(0, 128)],
                        out_hbm.at[w, e_hi, wid * JJ + sl], wsem[b])

        def drain_write(w, b):
            for e_hi in range(D // 8):
                for sl in range(JJ):
                    pltpu.make_async_copy(
                        stage_v.at[b, e_hi, sl, pl.ds(0, 8), pl.ds(0, 128)],
                        out_hbm.at[w, e_hi, wid * JJ + sl], wsem[b]).wait()

        fire_idx(0, 0)
        wait_idx(0, 0)
        fire_gather(0)
        fire_idx(1, 1)

        def outer(k, carry):
            for b in (0, 1):
                w = 2 * k + b
                drain_gather(b)
                if b == 0:
                    wait_idx(w + 1, 1)
                    fire_gather(1)
                else:
                    @pl.when(k < W // 2 - 1)
                    def _():
                        wait_idx(w + 1, 0)
                        fire_gather(0)

                @pl.when(k < W // 2 - 1)
                def _():
                    fire_idx(w + 2, b)

                @pl.when(k > 0)
                def _():
                    drain_write(w, b)

                transpose_scale(b)
                fire_write(w, b)
            return carry

        lax.fori_loop(0, W // 2, outer, 0)
        drain_write(W - 2, 0)
        drain_write(W - 1, 1)

    return emb


def kernel(input, table):
    idx3 = input.T.reshape(W, SH, 128)
    out5 = _emb()(table, idx3)
    return out5.transpose(2, 4, 0, 1, 3).reshape(S, W, D)
